# baseline (speedup 1.0000x reference)
import functools

import jax
import jax.numpy as jnp
from jax import lax
from jax.experimental import pallas as pl
from jax.experimental.pallas import tpu as pltpu
from jax.experimental.pallas import tpu_sc as plsc

N, E, D, H, R = 10000, 320000, 128, 128, 4

NC, NS, L = 2, 16, 16
NW = NC * NS
EPW = E // NW
CH = 16
NCH = EPW // CH
ROWS_PT = 640
MCH = 32
PK_SLAB = 10048
PK_PAD = 8 * 10048 + 24 * 9984 + 64

CNT_PT = 2560
CNT_PAD = NS * CNT_PT

BLK = 1000


def _rnorm(x):
  return jnp.clip(
      jnp.sqrt(jnp.sum(x * x, axis=-1, keepdims=True)), 1e-15, None)


def _artanh(x):
  x = jnp.clip(x, -1 + 1e-7, 1 - 1e-7)
  return 0.5 * jnp.log((1 + x) / (1 - x))


def _proj(x, sc):
  maxnorm = (1.0 - 1e-5) / sc
  n = _rnorm(x)
  return jnp.where(n > maxnorm, x / n * maxnorm, x)


def _expmap0(u, sc):
  n = _rnorm(u)
  return jnp.tanh(sc * n) * u / (sc * n)


def _logmap0(p, sc):
  n = _rnorm(p)
  return _artanh(sc * n) * p / (sc * n)


def _mobius_add(x, y, c):
  x2 = jnp.sum(x * x, -1, keepdims=True)
  y2 = jnp.sum(y * y, -1, keepdims=True)
  xy = jnp.sum(x * y, -1, keepdims=True)
  num = (1 + 2 * c * xy + c * y2) * x + (1 - c * x2) * y
  den = 1 + 2 * c * xy + c * c * x2 * y2
  return num / jnp.clip(den, 1e-15, None)


def _matT(x, M):
  return lax.dot_general(x, M, (((1,), (1,)), ((), ())),
                         preferred_element_type=jnp.float32)


def _mobius_matvec(x, M, c, sc):
  xn = _rnorm(x)
  mx = _matT(x, M)
  mxn = _rnorm(mx)
  return jnp.tanh(mxn / xn * _artanh(sc * xn)) * mx / (mxn * sc)


def _hyp_linear(x, W, b, c, sc):
  mv = _proj(_mobius_matvec(x, W, c, sc), sc)
  bh = _proj(_expmap0(b, sc), sc)
  return _proj(_mobius_add(mv, bh, c), sc)


def _elu(x):
  return jnp.where(x > 0, x, jnp.exp(x) - 1.0)


def _film_pre(hl, inv, Ws, Fs, Wr, Fr, skip_ref, gb_ref, wx_ref):
  bg = jnp.dot(hl, Fs, preferred_element_type=jnp.float32)
  ws = _matT(hl, Ws)
  skip_ref[...] = jnp.maximum(bg[:, :H] * ws + bg[:, H:], 0.0)
  for r in range(R):
    bgr = jnp.dot(hl, Fr[r], preferred_element_type=jnp.float32)
    gb_ref[r] = bgr * inv[:, r:r + 1]
    wx_ref[r] = _matT(hl, Wr[r])


def _tc1_body(x_ref, cnt_ref, Win_ref, bin_ref, W1s_ref, F1s_ref, W1_ref,
              F1_ref, c0_ref, c1_ref, skip_ref, gb_ref, wx_ref, inv_ref):
  c0 = c0_ref[0, 0]
  c1 = c1_ref[0, 0]
  sc0 = jnp.sqrt(c0)
  sc1 = jnp.sqrt(c1)
  x = x_ref[...]
  h = _proj(_expmap0(x, sc0), sc0)
  h = _hyp_linear(h, Win_ref[...], bin_ref[...], c0, sc0)
  h = _proj(_expmap0(_elu(_logmap0(h, sc0)), sc1), sc1)
  hl = _logmap0(h, sc1)
  cnt = cnt_ref[0] + cnt_ref[1]
  inv = 1.0 / jnp.maximum(cnt, 1.0)
  inv_ref[...] = inv
  _film_pre(hl, inv, W1s_ref[...], F1s_ref[...], W1_ref, F1_ref,
            skip_ref, gb_ref, wx_ref)


def _tc2_body(skip_ref, acc_ref, inv_ref, W2s_ref, F2s_ref, W2_ref, F2_ref,
              c1_ref, c2_ref, skip2_ref, gb_ref, wx_ref):
  c1 = c1_ref[0, 0]
  c2 = c2_ref[0, 0]
  sc1 = jnp.sqrt(c1)
  sc2 = jnp.sqrt(c2)
  out1 = skip_ref[...] + acc_ref[0] + acc_ref[1]
  h = _proj(_expmap0(out1, sc1), sc1)
  hl = _logmap0(h, sc2)
  _film_pre(hl, inv_ref[...], W2s_ref[...], F2s_ref[...], W2_ref, F2_ref,
            skip2_ref, gb_ref, wx_ref)


def _tc3_body(skip_ref, acc_ref, Wout_ref, bout_ref, c2_ref, c3_ref, y_ref):
  c2 = c2_ref[0, 0]
  c3 = c3_ref[0, 0]
  sc2 = jnp.sqrt(c2)
  sc3 = jnp.sqrt(c3)
  out2 = skip_ref[...] + acc_ref[0] + acc_ref[1]
  h = _proj(_expmap0(out2, sc2), sc2)
  h = _hyp_linear(h, Wout_ref[...], bout_ref[...], c3, sc3)
  y_ref[...] = _logmap0(h, sc3)


def _unpack(p):
  s = p & 0x3FFF
  d = lax.shift_right_logical(p, 14) & 0x3FFF
  e = lax.shift_right_logical(p, 28)
  return s, d, e


def _sc_counts_body(pk_hbm, zc_hbm, out_hbm, pk_v, ci_v, ones_v, buf_v, cacc):
  cid = lax.axis_index("c")
  sid = lax.axis_index("s")
  wid = sid * NC + cid
  base = wid * EPW

  pltpu.sync_copy(zc_hbm, buf_v)
  pltpu.sync_copy(buf_v, cacc.at[pl.ds(sid * CNT_PT, CNT_PT)])
  plsc.subcore_barrier()

  pltpu.sync_copy(pk_hbm.at[pl.ds(base, EPW)], pk_v)
  ones_v[...] = jnp.ones((CH,), jnp.float32)

  def body(i, carry):
    _, d, e = _unpack(pk_v[pl.ds(i * CH, CH)])
    ci_v[...] = d * R + e
    pltpu.sync_copy(ones_v, cacc.at[ci_v], add=True)
    return carry

  lax.fori_loop(0, NCH, body, 0)
  plsc.subcore_barrier()

  pltpu.sync_copy(cacc.at[pl.ds(sid * CNT_PT, CNT_PT)], buf_v)
  pltpu.sync_copy(buf_v, out_hbm.at[cid, pl.ds(sid * CNT_PT, CNT_PT)])


def _sc_msg_body(gb_hbm, wx_hbm, pk_hbm, z_hbm, out_hbm,
                 pk_v,
                 gbi0, wxi0, di0, di0b, gbi1, wxi1, di1,
                 gbr0, wxr0, msg0, gbr1, wxr1, msg1,
                 d16, acc, sem0, sem1, sem2, sem3, sem4, sem5):
  cid = lax.axis_index("c")
  sid = lax.axis_index("s")
  wid = sid * NC + cid
  base = jnp.where(wid < 8, wid * PK_SLAB, 8 * PK_SLAB + (wid - 8) * 9984)
  npairs = jnp.where(wid < 8, 157, 156)

  pltpu.sync_copy(z_hbm, msg0)
  pltpu.sync_copy(z_hbm.at[pl.ds(0, 16), :], d16)
  for k in range(ROWS_PT // MCH):
    r0 = sid * ROWS_PT + k * MCH

    @pl.when(r0 + MCH <= N)
    def _():
      pltpu.sync_copy(msg0, acc.at[pl.ds(r0, MCH), :])

  @pl.when(sid == NS - 1)
  def _():
    pltpu.sync_copy(d16, acc.at[pl.ds(N - 16, 16), :])

  plsc.subcore_barrier()

  pltpu.sync_copy(pk_hbm.at[pl.ds(base, PK_SLAB)], pk_v)

  def build(off, gbi, wxi, di):
    for g in range(MCH // L):
      s, d, e = _unpack(pk_v[pl.ds(off + g * L, L)])
      rbase = e * N
      gbi[pl.ds(g * L, L)] = rbase + d
      wxi[pl.ds(g * L, L)] = rbase + s
      di[pl.ds(g * L, L)] = d

  UB = 4

  def compute(gbr, wxr, msg):
    @plsc.parallel_loop(0, MCH // UB, unroll=4)
    def _(sb):
      for e in range(UB):
        ei = sb * UB + e
        for j in range(H // L):
          g = gbr[ei, pl.ds(j * L, L)]
          b = gbr[ei, pl.ds(H + j * L, L)]
          w = wxr[ei, pl.ds(j * L, L)]
          msg[ei, pl.ds(j * L, L)] = jnp.maximum(g * w + b, 0.0)

  build(0, gbi0, wxi0, di0)
  pltpu.async_copy(gb_hbm.at[gbi0], gbr0, sem0)
  pltpu.async_copy(wx_hbm.at[wxi0], wxr0, sem1)

  def pair(k, carry):
    build(k * 2 * MCH + MCH, gbi1, wxi1, di1)
    h1a = pltpu.async_copy(gb_hbm.at[gbi1], gbr1, sem2)
    h1b = pltpu.async_copy(wx_hbm.at[wxi1], wxr1, sem3)
    pltpu.make_async_copy(gb_hbm.at[gbi0], gbr0, sem0).wait()
    pltpu.make_async_copy(wx_hbm.at[wxi0], wxr0, sem1).wait()
    compute(gbr0, wxr0, msg0)
    for g in range(MCH // L):
      di0b[pl.ds(g * L, L)] = di0[pl.ds(g * L, L)]
    s0 = pltpu.async_copy(msg0, acc.at[di0b], sem4, add=True)

    @pl.when(k + 1 < npairs)
    def _():
      build((k + 1) * 2 * MCH, gbi0, wxi0, di0)
      pltpu.async_copy(gb_hbm.at[gbi0], gbr0, sem0)
      pltpu.async_copy(wx_hbm.at[wxi0], wxr0, sem1)

    h1a.wait()
    h1b.wait()
    compute(gbr1, wxr1, msg1)
    s1 = pltpu.async_copy(msg1, acc.at[di1], sem5, add=True)
    s0.wait()
    s1.wait()
    return carry

  lax.fori_loop(0, npairs, pair, 0)

  plsc.subcore_barrier()

  for k in range(ROWS_PT // MCH):
    r0 = sid * ROWS_PT + k * MCH

    @pl.when(r0 + MCH <= N)
    def _():
      pltpu.sync_copy(acc.at[pl.ds(r0, MCH), :], msg0)
      pltpu.sync_copy(msg0, out_hbm.at[cid, pl.ds(r0, MCH), :])

  @pl.when(sid == NS - 1)
  def _():
    pltpu.sync_copy(acc.at[pl.ds(N - 16, 16), :], d16)
    pltpu.sync_copy(d16, out_hbm.at[cid, pl.ds(N - 16, 16), :])


_SC_KERNELS = None


def _get_sc_kernels():
  global _SC_KERNELS
  if _SC_KERNELS is None:
    mesh = plsc.VectorSubcoreMesh(
        core_axis_name="c", subcore_axis_name="s",
        num_cores=NC, num_subcores=NS)
    counts = pl.kernel(
        _sc_counts_body,
        out_type=jax.ShapeDtypeStruct((NC, CNT_PAD), jnp.float32),
        mesh=mesh,
        scratch_types=[
            pltpu.VMEM((EPW,), jnp.int32),
            pltpu.VMEM((CH,), jnp.int32),
            pltpu.VMEM((CH,), jnp.float32),
            pltpu.VMEM((CNT_PT,), jnp.float32),
            pltpu.VMEM_SHARED((CNT_PAD,), jnp.float32),
        ],
    )
    msg = pl.kernel(
        _sc_msg_body,
        out_type=jax.ShapeDtypeStruct((NC, N, H), jnp.float32),
        mesh=mesh,
        scratch_types=[
            pltpu.VMEM((PK_SLAB,), jnp.int32),
            pltpu.VMEM((MCH,), jnp.int32),
            pltpu.VMEM((MCH,), jnp.int32),
            pltpu.VMEM((MCH,), jnp.int32),
            pltpu.VMEM((MCH,), jnp.int32),
            pltpu.VMEM((MCH,), jnp.int32),
            pltpu.VMEM((MCH,), jnp.int32),
            pltpu.VMEM((MCH,), jnp.int32),
            pltpu.VMEM((MCH, 2 * H), jnp.float32),
            pltpu.VMEM((MCH, H), jnp.float32),
            pltpu.VMEM((MCH, H), jnp.float32),
            pltpu.VMEM((MCH, 2 * H), jnp.float32),
            pltpu.VMEM((MCH, H), jnp.float32),
            pltpu.VMEM((MCH, H), jnp.float32),
            pltpu.VMEM((16, H), jnp.float32),
            pltpu.VMEM_SHARED((N, H), jnp.float32),
            pltpu.SemaphoreType.DMA,
            pltpu.SemaphoreType.DMA,
            pltpu.SemaphoreType.DMA,
            pltpu.SemaphoreType.DMA,
            pltpu.SemaphoreType.DMA,
            pltpu.SemaphoreType.DMA,
        ],
    )
    _SC_KERNELS = (counts, msg)
  return _SC_KERNELS


def _full(shape):
  return pl.BlockSpec(shape, lambda i: (0,) * len(shape))


def _rows(width):
  return pl.BlockSpec((BLK, width), lambda i: (i, 0))


_REL3 = pl.BlockSpec((R, BLK, 2 * H), lambda i: (0, i, 0))
_REL3W = pl.BlockSpec((R, BLK, H), lambda i: (0, i, 0))
_ACC3 = pl.BlockSpec((NC, BLK, H), lambda i: (0, i, 0))


def _tc1(x, cnt, W_in, b_in, W1s, F1s, W1, F1, c0, c1):
  return pl.pallas_call(
      _tc1_body,
      grid=(N // BLK,),
      in_specs=[
          _rows(D),
          pl.BlockSpec((NC, BLK, R), lambda i: (0, i, 0)),
          _full((H, D)), _full((1, H)),
          _full((H, H)), _full((H, 2 * H)),
          _full((R, H, H)), _full((R, H, 2 * H)),
          _full((1, 1)), _full((1, 1)),
      ],
      out_specs=[_rows(H), _REL3, _REL3W, _rows(R)],
      out_shape=[
          jax.ShapeDtypeStruct((N, H), jnp.float32),
          jax.ShapeDtypeStruct((R, N, 2 * H), jnp.float32),
          jax.ShapeDtypeStruct((R, N, H), jnp.float32),
          jax.ShapeDtypeStruct((N, R), jnp.float32),
      ],
  )(x, cnt, W_in, b_in, W1s, F1s, W1, F1, c0, c1)


def _tc2(skip1, acc1, inv, W2s, F2s, W2, F2, c1, c2):
  return pl.pallas_call(
      _tc2_body,
      grid=(N // BLK,),
      in_specs=[
          _rows(H), _ACC3, _rows(R),
          _full((H, H)), _full((H, 2 * H)),
          _full((R, H, H)), _full((R, H, 2 * H)),
          _full((1, 1)), _full((1, 1)),
      ],
      out_specs=[_rows(H), _REL3, _REL3W],
      out_shape=[
          jax.ShapeDtypeStruct((N, H), jnp.float32),
          jax.ShapeDtypeStruct((R, N, 2 * H), jnp.float32),
          jax.ShapeDtypeStruct((R, N, H), jnp.float32),
      ],
  )(skip1, acc1, inv, W2s, F2s, W2, F2, c1, c2)


def _tc3(skip2, acc2, W_out, b_out, c2, c3):
  return pl.pallas_call(
      _tc3_body,
      grid=(N // BLK,),
      in_specs=[
          _rows(H), _ACC3,
          _full((D, H)), _full((1, D)),
          _full((1, 1)), _full((1, 1)),
      ],
      out_specs=_rows(D),
      out_shape=jax.ShapeDtypeStruct((N, D), jnp.float32),
  )(skip2, acc2, W_out, b_out, c2, c3)


def kernel(x, adj, edge_type, c0, c1, c2, c3, W_in, b_in, W1, F1, W1s, F1s,
           W2, F2, W2s, F2s, W_out, b_out):
  src = adj[0].astype(jnp.int32)
  dst = adj[1].astype(jnp.int32)
  et = edge_type.astype(jnp.int32)
  packed = src + (dst << 14) + (et << 28)
  packed = jnp.concatenate(
      [packed, jnp.zeros((PK_PAD - E,), jnp.int32)])
  c0r = c0.reshape(1, 1)
  c1r = c1.reshape(1, 1)
  c2r = c2.reshape(1, 1)
  c3r = c3.reshape(1, 1)

  zc = jnp.zeros((CNT_PT,), jnp.float32)
  z_rows = jnp.zeros((MCH, H), jnp.float32)

  _sc_counts, _sc_msg = _get_sc_kernels()
  cnt_pad = _sc_counts(packed, zc)
  cnt = cnt_pad[:, :N * R].reshape(NC, N, R)

  skip1, gb1, wx1, inv = _tc1(x, cnt, W_in, b_in.reshape(1, H),
                              W1s, F1s, W1, F1, c0r, c1r)
  acc1 = _sc_msg(gb1.reshape(R * N, 2 * H), wx1.reshape(R * N, H),
                 packed, z_rows)
  skip2, gb2, wx2 = _tc2(skip1, acc1, inv, W2s, F2s, W2, F2, c1r, c2r)
  acc2 = _sc_msg(gb2.reshape(R * N, 2 * H), wx2.reshape(R * N, H),
                 packed, z_rows)
  return _tc3(skip2, acc2, W_out, b_out.reshape(1, D), c2r, c3r)


# speedup vs baseline: 1.0015x; 1.0015x over previous
import functools

import jax
import jax.numpy as jnp
from jax import lax
from jax.experimental import pallas as pl
from jax.experimental.pallas import tpu as pltpu
from jax.experimental.pallas import tpu_sc as plsc

N, E, D, H, R = 10000, 320000, 128, 128, 4

NC, NS, L = 2, 16, 16
NW = NC * NS
EPW = E // NW
CH = 16
NCH = EPW // CH
ROWS_PT = 640
MCH = 32
PK_SLAB = 10048
PK_PAD = 8 * 10048 + 24 * 9984 + 64

CNT_PT = 2560
CNT_PAD = NS * CNT_PT

BLK = 1000


def _rnorm(x):
  return jnp.clip(
      jnp.sqrt(jnp.sum(x * x, axis=-1, keepdims=True)), 1e-15, None)


def _artanh(x):
  x = jnp.clip(x, -1 + 1e-7, 1 - 1e-7)
  return 0.5 * jnp.log((1 + x) / (1 - x))


def _proj(x, sc):
  maxnorm = (1.0 - 1e-5) / sc
  n = _rnorm(x)
  return jnp.where(n > maxnorm, x / n * maxnorm, x)


def _expmap0(u, sc):
  n = _rnorm(u)
  return jnp.tanh(sc * n) * u / (sc * n)


def _logmap0(p, sc):
  n = _rnorm(p)
  return _artanh(sc * n) * p / (sc * n)


def _mobius_add(x, y, c):
  x2 = jnp.sum(x * x, -1, keepdims=True)
  y2 = jnp.sum(y * y, -1, keepdims=True)
  xy = jnp.sum(x * y, -1, keepdims=True)
  num = (1 + 2 * c * xy + c * y2) * x + (1 - c * x2) * y
  den = 1 + 2 * c * xy + c * c * x2 * y2
  return num / jnp.clip(den, 1e-15, None)


def _matT(x, M):
  return lax.dot_general(x, M, (((1,), (1,)), ((), ())),
                         preferred_element_type=jnp.float32)


def _mobius_matvec(x, M, c, sc):
  xn = _rnorm(x)
  mx = _matT(x, M)
  mxn = _rnorm(mx)
  return jnp.tanh(mxn / xn * _artanh(sc * xn)) * mx / (mxn * sc)


def _hyp_linear(x, W, b, c, sc):
  mv = _proj(_mobius_matvec(x, W, c, sc), sc)
  bh = _proj(_expmap0(b, sc), sc)
  return _proj(_mobius_add(mv, bh, c), sc)


def _elu(x):
  return jnp.where(x > 0, x, jnp.exp(x) - 1.0)


def _film_pre(hl, inv, Ws, Fs, Wr, Fr, skip_ref, gb_ref, wx_ref):
  bg = jnp.dot(hl, Fs, preferred_element_type=jnp.float32)
  ws = _matT(hl, Ws)
  skip_ref[...] = jnp.maximum(bg[:, :H] * ws + bg[:, H:], 0.0)
  for r in range(R):
    bgr = jnp.dot(hl, Fr[r], preferred_element_type=jnp.float32)
    gb_ref[r] = bgr * inv[:, r:r + 1]
    wx_ref[r] = _matT(hl, Wr[r])


def _tc1_body(x_ref, cnt_ref, Win_ref, bin_ref, W1s_ref, F1s_ref, W1_ref,
              F1_ref, c0_ref, c1_ref, skip_ref, gb_ref, wx_ref, inv_ref):
  c0 = c0_ref[0, 0]
  c1 = c1_ref[0, 0]
  sc0 = jnp.sqrt(c0)
  sc1 = jnp.sqrt(c1)
  x = x_ref[...]
  h = _proj(_expmap0(x, sc0), sc0)
  h = _hyp_linear(h, Win_ref[...], bin_ref[...], c0, sc0)
  h = _proj(_expmap0(_elu(_logmap0(h, sc0)), sc1), sc1)
  hl = _logmap0(h, sc1)
  cnt = cnt_ref[0] + cnt_ref[1]
  inv = 1.0 / jnp.maximum(cnt, 1.0)
  inv_ref[...] = inv
  _film_pre(hl, inv, W1s_ref[...], F1s_ref[...], W1_ref, F1_ref,
            skip_ref, gb_ref, wx_ref)


def _tc2_body(skip_ref, acc_ref, inv_ref, W2s_ref, F2s_ref, W2_ref, F2_ref,
              c1_ref, c2_ref, skip2_ref, gb_ref, wx_ref):
  c1 = c1_ref[0, 0]
  c2 = c2_ref[0, 0]
  sc1 = jnp.sqrt(c1)
  sc2 = jnp.sqrt(c2)
  out1 = skip_ref[...] + acc_ref[0] + acc_ref[1]
  h = _proj(_expmap0(out1, sc1), sc1)
  hl = _logmap0(h, sc2)
  _film_pre(hl, inv_ref[...], W2s_ref[...], F2s_ref[...], W2_ref, F2_ref,
            skip2_ref, gb_ref, wx_ref)


def _tc3_body(skip_ref, acc_ref, Wout_ref, bout_ref, c2_ref, c3_ref, y_ref):
  c2 = c2_ref[0, 0]
  c3 = c3_ref[0, 0]
  sc2 = jnp.sqrt(c2)
  sc3 = jnp.sqrt(c3)
  out2 = skip_ref[...] + acc_ref[0] + acc_ref[1]
  h = _proj(_expmap0(out2, sc2), sc2)
  h = _hyp_linear(h, Wout_ref[...], bout_ref[...], c3, sc3)
  y_ref[...] = _logmap0(h, sc3)


def _unpack(p):
  s = p & 0x3FFF
  d = lax.shift_right_logical(p, 14) & 0x3FFF
  e = lax.shift_right_logical(p, 28)
  return s, d, e


def _sc_counts_body(pk_hbm, zc_hbm, out_hbm, pk_v, ci_v, ones_v, buf_v, cacc):
  cid = lax.axis_index("c")
  sid = lax.axis_index("s")
  wid = sid * NC + cid
  base = wid * EPW

  pltpu.sync_copy(zc_hbm, buf_v)
  pltpu.sync_copy(buf_v, cacc.at[pl.ds(sid * CNT_PT, CNT_PT)])
  plsc.subcore_barrier()

  pltpu.sync_copy(pk_hbm.at[pl.ds(base, EPW)], pk_v)
  ones_v[...] = jnp.ones((CH,), jnp.float32)

  def body(i, carry):
    _, d, e = _unpack(pk_v[pl.ds(i * CH, CH)])
    ci_v[...] = d * R + e
    pltpu.sync_copy(ones_v, cacc.at[ci_v], add=True)
    return carry

  lax.fori_loop(0, NCH, body, 0)
  plsc.subcore_barrier()

  pltpu.sync_copy(cacc.at[pl.ds(sid * CNT_PT, CNT_PT)], buf_v)
  pltpu.sync_copy(buf_v, out_hbm.at[cid, pl.ds(sid * CNT_PT, CNT_PT)])


def _sc_msg_body(gb_hbm, wx_hbm, pk_hbm, z_hbm, out_hbm,
                 pk_v,
                 gbi0, wxi0, di0, di0b, gbi1, wxi1, di1,
                 gbr0, wxr0, msg0, gbr1, wxr1, msg1,
                 d16, acc, sem0, sem1, sem2, sem3, sem4, sem5):
  cid = lax.axis_index("c")
  sid = lax.axis_index("s")
  wid = sid * NC + cid
  base = jnp.where(wid < 8, wid * PK_SLAB, 8 * PK_SLAB + (wid - 8) * 9984)
  npairs = jnp.where(wid < 8, 157, 156)

  pltpu.sync_copy(z_hbm, msg0)
  pltpu.sync_copy(z_hbm.at[pl.ds(0, 16), :], d16)
  for k in range(ROWS_PT // MCH):
    r0 = sid * ROWS_PT + k * MCH

    @pl.when(r0 + MCH <= N)
    def _():
      pltpu.sync_copy(msg0, acc.at[pl.ds(r0, MCH), :])

  @pl.when(sid == NS - 1)
  def _():
    pltpu.sync_copy(d16, acc.at[pl.ds(N - 16, 16), :])

  plsc.subcore_barrier()

  pltpu.sync_copy(pk_hbm.at[pl.ds(base, PK_SLAB)], pk_v)

  def build(off, gbi, wxi, di):
    for g in range(MCH // L):
      s, d, e = _unpack(pk_v[pl.ds(off + g * L, L)])
      rbase = e * N
      gbi[pl.ds(g * L, L)] = rbase + d
      wxi[pl.ds(g * L, L)] = rbase + s
      di[pl.ds(g * L, L)] = d

  UB = 4

  def compute(gbr, wxr, msg):
    @plsc.parallel_loop(0, MCH // UB, unroll=2)
    def _(sb):
      for e in range(UB):
        ei = sb * UB + e
        for j in range(H // L):
          g = gbr[ei, pl.ds(j * L, L)]
          b = gbr[ei, pl.ds(H + j * L, L)]
          w = wxr[ei, pl.ds(j * L, L)]
          msg[ei, pl.ds(j * L, L)] = jnp.maximum(g * w + b, 0.0)

  build(0, gbi0, wxi0, di0)
  pltpu.async_copy(gb_hbm.at[gbi0], gbr0, sem0)
  pltpu.async_copy(wx_hbm.at[wxi0], wxr0, sem1)

  def pair(k, carry):
    build(k * 2 * MCH + MCH, gbi1, wxi1, di1)
    h1a = pltpu.async_copy(gb_hbm.at[gbi1], gbr1, sem2)
    h1b = pltpu.async_copy(wx_hbm.at[wxi1], wxr1, sem3)
    pltpu.make_async_copy(gb_hbm.at[gbi0], gbr0, sem0).wait()
    pltpu.make_async_copy(wx_hbm.at[wxi0], wxr0, sem1).wait()
    compute(gbr0, wxr0, msg0)
    for g in range(MCH // L):
      di0b[pl.ds(g * L, L)] = di0[pl.ds(g * L, L)]
    s0 = pltpu.async_copy(msg0, acc.at[di0b], sem4, add=True)

    @pl.when(k + 1 < npairs)
    def _():
      build((k + 1) * 2 * MCH, gbi0, wxi0, di0)
      pltpu.async_copy(gb_hbm.at[gbi0], gbr0, sem0)
      pltpu.async_copy(wx_hbm.at[wxi0], wxr0, sem1)

    h1a.wait()
    h1b.wait()
    compute(gbr1, wxr1, msg1)
    s1 = pltpu.async_copy(msg1, acc.at[di1], sem5, add=True)
    s0.wait()
    s1.wait()
    return carry

  lax.fori_loop(0, npairs, pair, 0)

  plsc.subcore_barrier()

  for k in range(ROWS_PT // MCH):
    r0 = sid * ROWS_PT + k * MCH

    @pl.when(r0 + MCH <= N)
    def _():
      pltpu.sync_copy(acc.at[pl.ds(r0, MCH), :], msg0)
      pltpu.sync_copy(msg0, out_hbm.at[cid, pl.ds(r0, MCH), :])

  @pl.when(sid == NS - 1)
  def _():
    pltpu.sync_copy(acc.at[pl.ds(N - 16, 16), :], d16)
    pltpu.sync_copy(d16, out_hbm.at[cid, pl.ds(N - 16, 16), :])


_SC_KERNELS = None


def _get_sc_kernels():
  global _SC_KERNELS
  if _SC_KERNELS is None:
    mesh = plsc.VectorSubcoreMesh(
        core_axis_name="c", subcore_axis_name="s",
        num_cores=NC, num_subcores=NS)
    counts = pl.kernel(
        _sc_counts_body,
        out_type=jax.ShapeDtypeStruct((NC, CNT_PAD), jnp.float32),
        mesh=mesh,
        scratch_types=[
            pltpu.VMEM((EPW,), jnp.int32),
            pltpu.VMEM((CH,), jnp.int32),
            pltpu.VMEM((CH,), jnp.float32),
            pltpu.VMEM((CNT_PT,), jnp.float32),
            pltpu.VMEM_SHARED((CNT_PAD,), jnp.float32),
        ],
    )
    msg = pl.kernel(
        _sc_msg_body,
        out_type=jax.ShapeDtypeStruct((NC, N, H), jnp.float32),
        mesh=mesh,
        scratch_types=[
            pltpu.VMEM((PK_SLAB,), jnp.int32),
            pltpu.VMEM((MCH,), jnp.int32),
            pltpu.VMEM((MCH,), jnp.int32),
            pltpu.VMEM((MCH,), jnp.int32),
            pltpu.VMEM((MCH,), jnp.int32),
            pltpu.VMEM((MCH,), jnp.int32),
            pltpu.VMEM((MCH,), jnp.int32),
            pltpu.VMEM((MCH,), jnp.int32),
            pltpu.VMEM((MCH, 2 * H), jnp.float32),
            pltpu.VMEM((MCH, H), jnp.float32),
            pltpu.VMEM((MCH, H), jnp.float32),
            pltpu.VMEM((MCH, 2 * H), jnp.float32),
            pltpu.VMEM((MCH, H), jnp.float32),
            pltpu.VMEM((MCH, H), jnp.float32),
            pltpu.VMEM((16, H), jnp.float32),
            pltpu.VMEM_SHARED((N, H), jnp.float32),
            pltpu.SemaphoreType.DMA,
            pltpu.SemaphoreType.DMA,
            pltpu.SemaphoreType.DMA,
            pltpu.SemaphoreType.DMA,
            pltpu.SemaphoreType.DMA,
            pltpu.SemaphoreType.DMA,
        ],
    )
    _SC_KERNELS = (counts, msg)
  return _SC_KERNELS


def _full(shape):
  return pl.BlockSpec(shape, lambda i: (0,) * len(shape))


def _rows(width):
  return pl.BlockSpec((BLK, width), lambda i: (i, 0))


_REL3 = pl.BlockSpec((R, BLK, 2 * H), lambda i: (0, i, 0))
_REL3W = pl.BlockSpec((R, BLK, H), lambda i: (0, i, 0))
_ACC3 = pl.BlockSpec((NC, BLK, H), lambda i: (0, i, 0))


def _tc1(x, cnt, W_in, b_in, W1s, F1s, W1, F1, c0, c1):
  return pl.pallas_call(
      _tc1_body,
      grid=(N // BLK,),
      in_specs=[
          _rows(D),
          pl.BlockSpec((NC, BLK, R), lambda i: (0, i, 0)),
          _full((H, D)), _full((1, H)),
          _full((H, H)), _full((H, 2 * H)),
          _full((R, H, H)), _full((R, H, 2 * H)),
          _full((1, 1)), _full((1, 1)),
      ],
      out_specs=[_rows(H), _REL3, _REL3W, _rows(R)],
      out_shape=[
          jax.ShapeDtypeStruct((N, H), jnp.float32),
          jax.ShapeDtypeStruct((R, N, 2 * H), jnp.float32),
          jax.ShapeDtypeStruct((R, N, H), jnp.float32),
          jax.ShapeDtypeStruct((N, R), jnp.float32),
      ],
  )(x, cnt, W_in, b_in, W1s, F1s, W1, F1, c0, c1)


def _tc2(skip1, acc1, inv, W2s, F2s, W2, F2, c1, c2):
  return pl.pallas_call(
      _tc2_body,
      grid=(N // BLK,),
      in_specs=[
          _rows(H), _ACC3, _rows(R),
          _full((H, H)), _full((H, 2 * H)),
          _full((R, H, H)), _full((R, H, 2 * H)),
          _full((1, 1)), _full((1, 1)),
      ],
      out_specs=[_rows(H), _REL3, _REL3W],
      out_shape=[
          jax.ShapeDtypeStruct((N, H), jnp.float32),
          jax.ShapeDtypeStruct((R, N, 2 * H), jnp.float32),
          jax.ShapeDtypeStruct((R, N, H), jnp.float32),
      ],
  )(skip1, acc1, inv, W2s, F2s, W2, F2, c1, c2)


def _tc3(skip2, acc2, W_out, b_out, c2, c3):
  return pl.pallas_call(
      _tc3_body,
      grid=(N // BLK,),
      in_specs=[
          _rows(H), _ACC3,
          _full((D, H)), _full((1, D)),
          _full((1, 1)), _full((1, 1)),
      ],
      out_specs=_rows(D),
      out_shape=jax.ShapeDtypeStruct((N, D), jnp.float32),
  )(skip2, acc2, W_out, b_out, c2, c3)


def kernel(x, adj, edge_type, c0, c1, c2, c3, W_in, b_in, W1, F1, W1s, F1s,
           W2, F2, W2s, F2s, W_out, b_out):
  src = adj[0].astype(jnp.int32)
  dst = adj[1].astype(jnp.int32)
  et = edge_type.astype(jnp.int32)
  packed = src + (dst << 14) + (et << 28)
  packed = jnp.concatenate(
      [packed, jnp.zeros((PK_PAD - E,), jnp.int32)])
  c0r = c0.reshape(1, 1)
  c1r = c1.reshape(1, 1)
  c2r = c2.reshape(1, 1)
  c3r = c3.reshape(1, 1)

  zc = jnp.zeros((CNT_PT,), jnp.float32)
  z_rows = jnp.zeros((MCH, H), jnp.float32)

  _sc_counts, _sc_msg = _get_sc_kernels()
  cnt_pad = _sc_counts(packed, zc)
  cnt = cnt_pad[:, :N * R].reshape(NC, N, R)

  skip1, gb1, wx1, inv = _tc1(x, cnt, W_in, b_in.reshape(1, H),
                              W1s, F1s, W1, F1, c0r, c1r)
  acc1 = _sc_msg(gb1.reshape(R * N, 2 * H), wx1.reshape(R * N, H),
                 packed, z_rows)
  skip2, gb2, wx2 = _tc2(skip1, acc1, inv, W2s, F2s, W2, F2, c1r, c2r)
  acc2 = _sc_msg(gb2.reshape(R * N, 2 * H), wx2.reshape(R * N, H),
                 packed, z_rows)
  return _tc3(skip2, acc2, W_out, b_out.reshape(1, D), c2r, c3r)


# speedup vs baseline: 1.0402x; 1.0386x over previous
import functools

import jax
import jax.numpy as jnp
from jax import lax
from jax.experimental import pallas as pl
from jax.experimental.pallas import tpu as pltpu
from jax.experimental.pallas import tpu_sc as plsc

N, E, D, H, R = 10000, 320000, 128, 128, 4

NC, NS, L = 2, 16, 16
NW = NC * NS
EPW = E // NW
CH = 16
NCH = EPW // CH
ROWS_PT = 640
MCH = 32
PK_SLAB = 10048
PK_PAD = 8 * 10048 + 24 * 9984 + 64

CCH = 128
CNT_PT = 2560
CNT_PAD = NS * CNT_PT

BLK = 1000


def _rnorm(x):
  return jnp.clip(
      jnp.sqrt(jnp.sum(x * x, axis=-1, keepdims=True)), 1e-15, None)


def _artanh(x):
  x = jnp.clip(x, -1 + 1e-7, 1 - 1e-7)
  return 0.5 * jnp.log((1 + x) / (1 - x))


def _proj(x, sc):
  maxnorm = (1.0 - 1e-5) / sc
  n = _rnorm(x)
  return jnp.where(n > maxnorm, x / n * maxnorm, x)


def _expmap0(u, sc):
  n = _rnorm(u)
  return jnp.tanh(sc * n) * u / (sc * n)


def _logmap0(p, sc):
  n = _rnorm(p)
  return _artanh(sc * n) * p / (sc * n)


def _mobius_add(x, y, c):
  x2 = jnp.sum(x * x, -1, keepdims=True)
  y2 = jnp.sum(y * y, -1, keepdims=True)
  xy = jnp.sum(x * y, -1, keepdims=True)
  num = (1 + 2 * c * xy + c * y2) * x + (1 - c * x2) * y
  den = 1 + 2 * c * xy + c * c * x2 * y2
  return num / jnp.clip(den, 1e-15, None)


def _matT(x, M):
  return lax.dot_general(x, M, (((1,), (1,)), ((), ())),
                         preferred_element_type=jnp.float32)


def _mobius_matvec(x, M, c, sc):
  xn = _rnorm(x)
  mx = _matT(x, M)
  mxn = _rnorm(mx)
  return jnp.tanh(mxn / xn * _artanh(sc * xn)) * mx / (mxn * sc)


def _hyp_linear(x, W, b, c, sc):
  mv = _proj(_mobius_matvec(x, W, c, sc), sc)
  bh = _proj(_expmap0(b, sc), sc)
  return _proj(_mobius_add(mv, bh, c), sc)


def _elu(x):
  return jnp.where(x > 0, x, jnp.exp(x) - 1.0)


def _film_pre(hl, inv, Ws, Fs, Wr, Fr, skip_ref, gb_ref, wx_ref):
  bg = jnp.dot(hl, Fs, preferred_element_type=jnp.float32)
  ws = _matT(hl, Ws)
  skip_ref[...] = jnp.maximum(bg[:, :H] * ws + bg[:, H:], 0.0)
  for r in range(R):
    bgr = jnp.dot(hl, Fr[r], preferred_element_type=jnp.float32)
    gb_ref[r] = bgr * inv[:, r:r + 1]
    wx_ref[r] = _matT(hl, Wr[r])


def _tc1_body(x_ref, cnt_ref, Win_ref, bin_ref, W1s_ref, F1s_ref, W1_ref,
              F1_ref, c0_ref, c1_ref, skip_ref, gb_ref, wx_ref, inv_ref):
  c0 = c0_ref[0, 0]
  c1 = c1_ref[0, 0]
  sc0 = jnp.sqrt(c0)
  sc1 = jnp.sqrt(c1)
  x = x_ref[...]
  h = _proj(_expmap0(x, sc0), sc0)
  h = _hyp_linear(h, Win_ref[...], bin_ref[...], c0, sc0)
  h = _proj(_expmap0(_elu(_logmap0(h, sc0)), sc1), sc1)
  hl = _logmap0(h, sc1)
  cnt = cnt_ref[0] + cnt_ref[1]
  inv = 1.0 / jnp.maximum(cnt, 1.0)
  inv_ref[...] = inv
  _film_pre(hl, inv, W1s_ref[...], F1s_ref[...], W1_ref, F1_ref,
            skip_ref, gb_ref, wx_ref)


def _tc2_body(skip_ref, acc_ref, inv_ref, W2s_ref, F2s_ref, W2_ref, F2_ref,
              c1_ref, c2_ref, skip2_ref, gb_ref, wx_ref):
  c1 = c1_ref[0, 0]
  c2 = c2_ref[0, 0]
  sc1 = jnp.sqrt(c1)
  sc2 = jnp.sqrt(c2)
  out1 = skip_ref[...] + acc_ref[0] + acc_ref[1]
  h = _proj(_expmap0(out1, sc1), sc1)
  hl = _logmap0(h, sc2)
  _film_pre(hl, inv_ref[...], W2s_ref[...], F2s_ref[...], W2_ref, F2_ref,
            skip2_ref, gb_ref, wx_ref)


def _tc3_body(skip_ref, acc_ref, Wout_ref, bout_ref, c2_ref, c3_ref, y_ref):
  c2 = c2_ref[0, 0]
  c3 = c3_ref[0, 0]
  sc2 = jnp.sqrt(c2)
  sc3 = jnp.sqrt(c3)
  out2 = skip_ref[...] + acc_ref[0] + acc_ref[1]
  h = _proj(_expmap0(out2, sc2), sc2)
  h = _hyp_linear(h, Wout_ref[...], bout_ref[...], c3, sc3)
  y_ref[...] = _logmap0(h, sc3)


def _unpack(p):
  s = p & 0x3FFF
  d = lax.shift_right_logical(p, 14) & 0x3FFF
  e = lax.shift_right_logical(p, 28)
  return s, d, e


def _sc_counts_body(pk_hbm, zc_hbm, out_hbm, pk_v, ci_v, ones_v, ci16_v,
                    ones16_v, buf_v, cacc):
  cid = lax.axis_index("c")
  sid = lax.axis_index("s")
  wid = sid * NC + cid
  base = wid * EPW

  pltpu.sync_copy(zc_hbm, buf_v)
  pltpu.sync_copy(buf_v, cacc.at[pl.ds(sid * CNT_PT, CNT_PT)])
  plsc.subcore_barrier()

  pltpu.sync_copy(pk_hbm.at[pl.ds(base, EPW)], pk_v)
  for g in range(CCH // L):
    ones_v[pl.ds(g * L, L)] = jnp.ones((L,), jnp.float32)

  def body(i, carry):
    for g in range(CCH // L):
      _, d, e = _unpack(pk_v[pl.ds(i * CCH + g * L, L)])
      ci_v[pl.ds(g * L, L)] = d * R + e
    pltpu.sync_copy(ones_v, cacc.at[ci_v], add=True)
    return carry

  lax.fori_loop(0, EPW // CCH, body, 0)
  ones16_v[...] = jnp.ones((L,), jnp.float32)
  _, d, e = _unpack(pk_v[pl.ds(EPW - L, L)])
  ci16_v[...] = d * R + e
  pltpu.sync_copy(ones16_v, cacc.at[ci16_v], add=True)
  plsc.subcore_barrier()

  pltpu.sync_copy(cacc.at[pl.ds(sid * CNT_PT, CNT_PT)], buf_v)
  pltpu.sync_copy(buf_v, out_hbm.at[cid, pl.ds(sid * CNT_PT, CNT_PT)])


def _sc_msg_body(gb_hbm, wx_hbm, pk_hbm, z_hbm, out_hbm,
                 pk_v,
                 gbi0, wxi0, di0, di0b, gbi1, wxi1, di1,
                 gbr0, wxr0, msg0, gbr1, wxr1, msg1,
                 d16, acc, sem0, sem1, sem2, sem3, sem4, sem5):
  cid = lax.axis_index("c")
  sid = lax.axis_index("s")
  wid = sid * NC + cid
  base = jnp.where(wid < 8, wid * PK_SLAB, 8 * PK_SLAB + (wid - 8) * 9984)
  npairs = jnp.where(wid < 8, 157, 156)

  pltpu.sync_copy(z_hbm, msg0)
  pltpu.sync_copy(z_hbm.at[pl.ds(0, 16), :], d16)
  for k in range(ROWS_PT // MCH):
    r0 = sid * ROWS_PT + k * MCH

    @pl.when(r0 + MCH <= N)
    def _():
      pltpu.sync_copy(msg0, acc.at[pl.ds(r0, MCH), :])

  @pl.when(sid == NS - 1)
  def _():
    pltpu.sync_copy(d16, acc.at[pl.ds(N - 16, 16), :])

  plsc.subcore_barrier()

  pltpu.sync_copy(pk_hbm.at[pl.ds(base, PK_SLAB)], pk_v)

  def build(off, gbi, wxi, di):
    for g in range(MCH // L):
      s, d, e = _unpack(pk_v[pl.ds(off + g * L, L)])
      rbase = e * N
      gbi[pl.ds(g * L, L)] = rbase + d
      wxi[pl.ds(g * L, L)] = rbase + s
      di[pl.ds(g * L, L)] = d

  UB = 4

  def compute(gbr, wxr, msg):
    @plsc.parallel_loop(0, MCH // UB, unroll=2)
    def _(sb):
      for e in range(UB):
        ei = sb * UB + e
        for j in range(H // L):
          g = gbr[ei, pl.ds(j * L, L)]
          b = gbr[ei, pl.ds(H + j * L, L)]
          w = wxr[ei, pl.ds(j * L, L)]
          msg[ei, pl.ds(j * L, L)] = jnp.maximum(g * w + b, 0.0)

  build(0, gbi0, wxi0, di0)
  pltpu.async_copy(gb_hbm.at[gbi0], gbr0, sem0)
  pltpu.async_copy(wx_hbm.at[wxi0], wxr0, sem1)

  def pair(k, carry):
    build(k * 2 * MCH + MCH, gbi1, wxi1, di1)
    h1a = pltpu.async_copy(gb_hbm.at[gbi1], gbr1, sem2)
    h1b = pltpu.async_copy(wx_hbm.at[wxi1], wxr1, sem3)
    pltpu.make_async_copy(gb_hbm.at[gbi0], gbr0, sem0).wait()
    pltpu.make_async_copy(wx_hbm.at[wxi0], wxr0, sem1).wait()
    compute(gbr0, wxr0, msg0)
    for g in range(MCH // L):
      di0b[pl.ds(g * L, L)] = di0[pl.ds(g * L, L)]
    s0 = pltpu.async_copy(msg0, acc.at[di0b], sem4, add=True)

    @pl.when(k + 1 < npairs)
    def _():
      build((k + 1) * 2 * MCH, gbi0, wxi0, di0)
      pltpu.async_copy(gb_hbm.at[gbi0], gbr0, sem0)
      pltpu.async_copy(wx_hbm.at[wxi0], wxr0, sem1)

    h1a.wait()
    h1b.wait()
    compute(gbr1, wxr1, msg1)
    s1 = pltpu.async_copy(msg1, acc.at[di1], sem5, add=True)
    s0.wait()
    s1.wait()
    return carry

  lax.fori_loop(0, npairs, pair, 0)

  plsc.subcore_barrier()

  for k in range(ROWS_PT // MCH):
    r0 = sid * ROWS_PT + k * MCH

    @pl.when(r0 + MCH <= N)
    def _():
      pltpu.sync_copy(acc.at[pl.ds(r0, MCH), :], msg0)
      pltpu.sync_copy(msg0, out_hbm.at[cid, pl.ds(r0, MCH), :])

  @pl.when(sid == NS - 1)
  def _():
    pltpu.sync_copy(acc.at[pl.ds(N - 16, 16), :], d16)
    pltpu.sync_copy(d16, out_hbm.at[cid, pl.ds(N - 16, 16), :])


_SC_KERNELS = None


def _get_sc_kernels():
  global _SC_KERNELS
  if _SC_KERNELS is None:
    mesh = plsc.VectorSubcoreMesh(
        core_axis_name="c", subcore_axis_name="s",
        num_cores=NC, num_subcores=NS)
    counts = pl.kernel(
        _sc_counts_body,
        out_type=jax.ShapeDtypeStruct((NC, CNT_PAD), jnp.float32),
        mesh=mesh,
        scratch_types=[
            pltpu.VMEM((EPW,), jnp.int32),
            pltpu.VMEM((CCH,), jnp.int32),
            pltpu.VMEM((CCH,), jnp.float32),
            pltpu.VMEM((L,), jnp.int32),
            pltpu.VMEM((L,), jnp.float32),
            pltpu.VMEM((CNT_PT,), jnp.float32),
            pltpu.VMEM_SHARED((CNT_PAD,), jnp.float32),
        ],
    )
    msg = pl.kernel(
        _sc_msg_body,
        out_type=jax.ShapeDtypeStruct((NC, N, H), jnp.float32),
        mesh=mesh,
        scratch_types=[
            pltpu.VMEM((PK_SLAB,), jnp.int32),
            pltpu.VMEM((MCH,), jnp.int32),
            pltpu.VMEM((MCH,), jnp.int32),
            pltpu.VMEM((MCH,), jnp.int32),
            pltpu.VMEM((MCH,), jnp.int32),
            pltpu.VMEM((MCH,), jnp.int32),
            pltpu.VMEM((MCH,), jnp.int32),
            pltpu.VMEM((MCH,), jnp.int32),
            pltpu.VMEM((MCH, 2 * H), jnp.float32),
            pltpu.VMEM((MCH, H), jnp.float32),
            pltpu.VMEM((MCH, H), jnp.float32),
            pltpu.VMEM((MCH, 2 * H), jnp.float32),
            pltpu.VMEM((MCH, H), jnp.float32),
            pltpu.VMEM((MCH, H), jnp.float32),
            pltpu.VMEM((16, H), jnp.float32),
            pltpu.VMEM_SHARED((N, H), jnp.float32),
            pltpu.SemaphoreType.DMA,
            pltpu.SemaphoreType.DMA,
            pltpu.SemaphoreType.DMA,
            pltpu.SemaphoreType.DMA,
            pltpu.SemaphoreType.DMA,
            pltpu.SemaphoreType.DMA,
        ],
    )
    _SC_KERNELS = (counts, msg)
  return _SC_KERNELS


def _full(shape):
  return pl.BlockSpec(shape, lambda i: (0,) * len(shape))


def _rows(width):
  return pl.BlockSpec((BLK, width), lambda i: (i, 0))


_REL3 = pl.BlockSpec((R, BLK, 2 * H), lambda i: (0, i, 0))
_REL3W = pl.BlockSpec((R, BLK, H), lambda i: (0, i, 0))
_ACC3 = pl.BlockSpec((NC, BLK, H), lambda i: (0, i, 0))


def _tc1(x, cnt, W_in, b_in, W1s, F1s, W1, F1, c0, c1):
  return pl.pallas_call(
      _tc1_body,
      grid=(N // BLK,),
      in_specs=[
          _rows(D),
          pl.BlockSpec((NC, BLK, R), lambda i: (0, i, 0)),
          _full((H, D)), _full((1, H)),
          _full((H, H)), _full((H, 2 * H)),
          _full((R, H, H)), _full((R, H, 2 * H)),
          _full((1, 1)), _full((1, 1)),
      ],
      out_specs=[_rows(H), _REL3, _REL3W, _rows(R)],
      out_shape=[
          jax.ShapeDtypeStruct((N, H), jnp.float32),
          jax.ShapeDtypeStruct((R, N, 2 * H), jnp.float32),
          jax.ShapeDtypeStruct((R, N, H), jnp.float32),
          jax.ShapeDtypeStruct((N, R), jnp.float32),
      ],
  )(x, cnt, W_in, b_in, W1s, F1s, W1, F1, c0, c1)


def _tc2(skip1, acc1, inv, W2s, F2s, W2, F2, c1, c2):
  return pl.pallas_call(
      _tc2_body,
      grid=(N // BLK,),
      in_specs=[
          _rows(H), _ACC3, _rows(R),
          _full((H, H)), _full((H, 2 * H)),
          _full((R, H, H)), _full((R, H, 2 * H)),
          _full((1, 1)), _full((1, 1)),
      ],
      out_specs=[_rows(H), _REL3, _REL3W],
      out_shape=[
          jax.ShapeDtypeStruct((N, H), jnp.float32),
          jax.ShapeDtypeStruct((R, N, 2 * H), jnp.float32),
          jax.ShapeDtypeStruct((R, N, H), jnp.float32),
      ],
  )(skip1, acc1, inv, W2s, F2s, W2, F2, c1, c2)


def _tc3(skip2, acc2, W_out, b_out, c2, c3):
  return pl.pallas_call(
      _tc3_body,
      grid=(N // BLK,),
      in_specs=[
          _rows(H), _ACC3,
          _full((D, H)), _full((1, D)),
          _full((1, 1)), _full((1, 1)),
      ],
      out_specs=_rows(D),
      out_shape=jax.ShapeDtypeStruct((N, D), jnp.float32),
  )(skip2, acc2, W_out, b_out, c2, c3)


def kernel(x, adj, edge_type, c0, c1, c2, c3, W_in, b_in, W1, F1, W1s, F1s,
           W2, F2, W2s, F2s, W_out, b_out):
  src = adj[0].astype(jnp.int32)
  dst = adj[1].astype(jnp.int32)
  et = edge_type.astype(jnp.int32)
  packed = src + (dst << 14) + (et << 28)
  packed = jnp.concatenate(
      [packed, jnp.zeros((PK_PAD - E,), jnp.int32)])
  c0r = c0.reshape(1, 1)
  c1r = c1.reshape(1, 1)
  c2r = c2.reshape(1, 1)
  c3r = c3.reshape(1, 1)

  zc = jnp.zeros((CNT_PT,), jnp.float32)
  z_rows = jnp.zeros((MCH, H), jnp.float32)

  _sc_counts, _sc_msg = _get_sc_kernels()
  cnt_pad = _sc_counts(packed, zc)
  cnt = cnt_pad[:, :N * R].reshape(NC, N, R)

  skip1, gb1, wx1, inv = _tc1(x, cnt, W_in, b_in.reshape(1, H),
                              W1s, F1s, W1, F1, c0r, c1r)
  acc1 = _sc_msg(gb1.reshape(R * N, 2 * H), wx1.reshape(R * N, H),
                 packed, z_rows)
  skip2, gb2, wx2 = _tc2(skip1, acc1, inv, W2s, F2s, W2, F2, c1r, c2r)
  acc2 = _sc_msg(gb2.reshape(R * N, 2 * H), wx2.reshape(R * N, H),
                 packed, z_rows)
  return _tc3(skip2, acc2, W_out, b_out.reshape(1, D), c2r, c3r)
